# Initial kernel scaffold; baseline (speedup 1.0000x reference)
import jax
import jax.numpy as jnp
from jax.experimental import pallas as pl


def kernel(h_user, h_item, W, src_idx, dst_idx):
    raise NotImplementedError("write your pallas kernel here")


# speedup vs baseline: 2.6141x; 2.6141x over previous
import functools

import jax
import jax.numpy as jnp
from jax import lax
from jax.experimental import pallas as pl
from jax.experimental.pallas import tpu as pltpu
from jax.experimental.pallas import tpu_sc as plsc

D = 128
NC = 2
NS = 16
NW = NC * NS
CHUNK = 80
LANES = 16


def _transform_table(h, w):
    rows, d = h.shape
    blk = 2000
    assert rows % blk == 0

    def body(x_ref, w_ref, o_ref):
        o_ref[...] = lax.dot_general(
            x_ref[...], w_ref[...],
            dimension_numbers=(((1,), (1,)), ((), ())),
            preferred_element_type=jnp.float32)

    return pl.pallas_call(
        body,
        grid=(rows // blk,),
        in_specs=[
            pl.BlockSpec((blk, d), lambda i: (i, 0)),
            pl.BlockSpec((d, d), lambda i: (0, 0)),
        ],
        out_specs=pl.BlockSpec((blk, d), lambda i: (i, 0)),
        out_shape=jax.ShapeDtypeStruct((rows, d), jnp.float32),
    )(h, w)


def _make_sc_scorer(e_total):
    per_w = e_total // NW
    n_chunks = per_w // CHUNK
    groups = CHUNK // LANES
    mesh = plsc.VectorSubcoreMesh(core_axis_name="c", subcore_axis_name="s")

    @functools.partial(
        pl.kernel,
        mesh=mesh,
        compiler_params=pltpu.CompilerParams(needs_layout_passes=False),
        out_type=jax.ShapeDtypeStruct((e_total,), jnp.float32),
        scratch_types=[
            pltpu.VMEM((per_w,), jnp.int32),
            pltpu.VMEM((per_w,), jnp.int32),
            pltpu.VMEM((per_w,), jnp.float32),
            pltpu.VMEM((CHUNK, D), jnp.float32),
            pltpu.VMEM((CHUNK, D), jnp.float32),
            pltpu.SemaphoreType.DMA,
        ],
    )
    def scorer(hu_t, hi, src_hbm, dst_hbm, out_hbm,
               sidx_v, didx_v, out_v, srows, drows, sem):
        wid = lax.axis_index("s") * NC + lax.axis_index("c")
        base = wid * per_w
        pltpu.sync_copy(src_hbm.at[pl.ds(base, per_w)], sidx_v)
        pltpu.sync_copy(dst_hbm.at[pl.ds(base, per_w)], didx_v)

        lane_iota = lax.iota(jnp.int32, LANES)

        def compute_chunk(c, s_ref, d_ref):
            def group_body(g, _):
                res = jnp.zeros((LANES,), jnp.float32)
                for j in range(LANES):
                    accs = [s_ref[g * LANES + j, pl.ds(k * LANES, LANES)]
                            * d_ref[g * LANES + j, pl.ds(k * LANES, LANES)]
                            for k in range(D // LANES)]
                    acc = ((accs[0] + accs[1]) + (accs[2] + accs[3])) + \
                          ((accs[4] + accs[5]) + (accs[6] + accs[7]))
                    res = jnp.where(lane_iota == j, jnp.sum(acc), res)
                out_v[pl.ds(c * CHUNK + g * LANES, LANES)] = res
                return 0
            lax.fori_loop(0, groups, group_body, 0)

        def chunk_body(c, _):
            off = c * CHUNK
            cp_s = pltpu.async_copy(hu_t.at[sidx_v.at[pl.ds(off, CHUNK)]], srows, sem)
            cp_d = pltpu.async_copy(hi.at[didx_v.at[pl.ds(off, CHUNK)]], drows, sem)
            cp_s.wait()
            cp_d.wait()
            compute_chunk(c, srows, drows)
            return 0

        lax.fori_loop(0, n_chunks, chunk_body, 0)
        pltpu.sync_copy(out_v, out_hbm.at[pl.ds(base, per_w)])

    return scorer


def kernel(h_user, h_item, W, src_idx, dst_idx):
    hu_t = _transform_table(h_user, W)
    scorer = _make_sc_scorer(src_idx.shape[0])
    return scorer(hu_t, h_item, src_idx, dst_idx)


# speedup vs baseline: 3.5391x; 1.3539x over previous
import functools

import jax
import jax.numpy as jnp
from jax import lax
from jax.experimental import pallas as pl
from jax.experimental.pallas import tpu as pltpu
from jax.experimental.pallas import tpu_sc as plsc

D = 128
NC = 2
NS = 16
NW = NC * NS
CHUNK = 80
LANES = 16


def _transform_table(h, w):
    rows, d = h.shape
    blk = 2000
    assert rows % blk == 0

    def body(x_ref, w_ref, o_ref):
        o_ref[...] = lax.dot_general(
            x_ref[...], w_ref[...],
            dimension_numbers=(((1,), (1,)), ((), ())),
            preferred_element_type=jnp.float32)

    return pl.pallas_call(
        body,
        grid=(rows // blk,),
        in_specs=[
            pl.BlockSpec((blk, d), lambda i: (i, 0)),
            pl.BlockSpec((d, d), lambda i: (0, 0)),
        ],
        out_specs=pl.BlockSpec((blk, d), lambda i: (i, 0)),
        out_shape=jax.ShapeDtypeStruct((rows, d), jnp.float32),
    )(h, w)


def _make_sc_scorer(e_total):
    per_w = e_total // NW
    n_chunks = per_w // CHUNK
    groups = CHUNK // LANES
    mesh = plsc.VectorSubcoreMesh(core_axis_name="c", subcore_axis_name="s")

    @functools.partial(
        pl.kernel,
        mesh=mesh,
        compiler_params=pltpu.CompilerParams(needs_layout_passes=False),
        out_type=jax.ShapeDtypeStruct((e_total,), jnp.float32),
        scratch_types=[
            pltpu.VMEM((per_w,), jnp.int32),
            pltpu.VMEM((per_w,), jnp.int32),
            pltpu.VMEM((per_w,), jnp.float32),
            pltpu.VMEM((CHUNK, D), jnp.float32),
            pltpu.VMEM((CHUNK, D), jnp.float32),
            pltpu.VMEM((CHUNK, D), jnp.float32),
            pltpu.VMEM((CHUNK, D), jnp.float32),
            pltpu.SemaphoreType.DMA,
            pltpu.SemaphoreType.DMA,
        ],
    )
    def scorer(hu_t, hi, src_hbm, dst_hbm, out_hbm,
               sidx_v, didx_v, out_v, srows_a, drows_a, srows_b, drows_b,
               sem_a, sem_b):
        wid = lax.axis_index("s") * NC + lax.axis_index("c")
        base = wid * per_w
        pltpu.sync_copy(src_hbm.at[pl.ds(base, per_w)], sidx_v)
        pltpu.sync_copy(dst_hbm.at[pl.ds(base, per_w)], didx_v)

        def fire(c, s_buf, d_buf, sem):
            off = c * CHUNK
            pltpu.async_copy(hu_t.at[sidx_v.at[pl.ds(off, CHUNK)]], s_buf, sem)
            pltpu.async_copy(hi.at[didx_v.at[pl.ds(off, CHUNK)]], d_buf, sem)

        def drain(s_buf, d_buf, sem):
            pltpu.make_async_copy(hu_t.at[sidx_v.at[pl.ds(0, CHUNK)]], s_buf, sem).wait()
            pltpu.make_async_copy(hi.at[didx_v.at[pl.ds(0, CHUNK)]], d_buf, sem).wait()

        lane_iota = lax.iota(jnp.int32, LANES)

        def compute_chunk(c, s_ref, d_ref):
            def group_body(g, _):
                res = jnp.zeros((LANES,), jnp.float32)
                for j in range(LANES):
                    accs = [s_ref[g * LANES + j, pl.ds(k * LANES, LANES)]
                            * d_ref[g * LANES + j, pl.ds(k * LANES, LANES)]
                            for k in range(D // LANES)]
                    acc = ((accs[0] + accs[1]) + (accs[2] + accs[3])) + \
                          ((accs[4] + accs[5]) + (accs[6] + accs[7]))
                    res = jnp.where(lane_iota == j, jnp.sum(acc), res)
                out_v[pl.ds(c * CHUNK + g * LANES, LANES)] = res
                return 0
            lax.fori_loop(0, groups, group_body, 0)

        assert n_chunks % 2 == 1
        fire(0, srows_a, drows_a, sem_a)

        def pair_body(p, _):
            c0 = 2 * p
            drain(srows_a, drows_a, sem_a)
            fire(c0 + 1, srows_b, drows_b, sem_b)
            compute_chunk(c0, srows_a, drows_a)
            drain(srows_b, drows_b, sem_b)
            fire(c0 + 2, srows_a, drows_a, sem_a)
            compute_chunk(c0 + 1, srows_b, drows_b)
            return 0

        lax.fori_loop(0, (n_chunks - 1) // 2, pair_body, 0)
        drain(srows_a, drows_a, sem_a)
        compute_chunk(n_chunks - 1, srows_a, drows_a)
        pltpu.sync_copy(out_v, out_hbm.at[pl.ds(base, per_w)])

    return scorer


def kernel(h_user, h_item, W, src_idx, dst_idx):
    hu_t = _transform_table(h_user, W)
    scorer = _make_sc_scorer(src_idx.shape[0])
    return scorer(hu_t, h_item, src_idx, dst_idx)
